# baseline (speedup 1.0000x reference)
import functools

import jax
import jax.numpy as jnp
from jax import lax
from jax.experimental import pallas as pl
from jax.experimental.pallas import tpu as pltpu
from jax.experimental.pallas import tpu_sc as plsc

BATCH = 4096
HIST = 200
EMBED_DIM = 64
B_TOTAL = BATCH * HIST

_info = plsc.get_sparse_core_info()
NUM_CORES = _info.num_cores
NUM_SUBCORES = _info.num_subcores
NW = NUM_CORES * NUM_SUBCORES
B_PER_W = B_TOTAL // NW

CHUNK = 1024
NCHUNK = B_PER_W // CHUNK


def _gather_body(table_hbm, idx_hbm, out_hbm, idx_v, rows_v, sem):
    c = lax.axis_index("c")
    s = lax.axis_index("s")
    wid = s * NUM_CORES + c
    base = wid * B_PER_W
    pltpu.sync_copy(idx_hbm.at[pl.ds(base, B_PER_W)], idx_v)

    def body(g, carry):
        off = pl.multiple_of(g * CHUNK, CHUNK)
        pltpu.async_copy(
            table_hbm.at[idx_v.at[pl.ds(off, CHUNK)]], rows_v, sem
        ).wait()
        pltpu.sync_copy(rows_v, out_hbm.at[pl.ds(base + off, CHUNK)])
        return carry

    lax.fori_loop(0, NCHUNK, body, 0)


@jax.jit
def _embedding_lookup(idx_flat, weight):
    mesh = plsc.VectorSubcoreMesh(core_axis_name="c", subcore_axis_name="s")
    fn = pl.kernel(
        _gather_body,
        out_type=jax.ShapeDtypeStruct((B_TOTAL, EMBED_DIM), jnp.float32),
        mesh=mesh,
        scratch_types=[
            pltpu.VMEM((B_PER_W,), jnp.int32),
            pltpu.VMEM((CHUNK, EMBED_DIM), jnp.float32),
            pltpu.SemaphoreType.DMA,
        ],
        compiler_params=pltpu.CompilerParams(use_tc_tiling_on_sc=False),
    )
    return fn(weight, idx_flat)


def kernel(input, weight):
    idx_flat = input.reshape(-1).astype(jnp.int32)
    out = _embedding_lookup(idx_flat, weight)
    return out.reshape(BATCH, HIST, EMBED_DIM)


# speedup vs baseline: 1.0042x; 1.0042x over previous
import functools

import jax
import jax.numpy as jnp
from jax import lax
from jax.experimental import pallas as pl
from jax.experimental.pallas import tpu as pltpu
from jax.experimental.pallas import tpu_sc as plsc

BATCH = 4096
HIST = 200
EMBED_DIM = 64
B_TOTAL = BATCH * HIST

_info = plsc.get_sparse_core_info()
NUM_CORES = _info.num_cores
NUM_SUBCORES = _info.num_subcores
NW = NUM_CORES * NUM_SUBCORES
B_PER_W = B_TOTAL // NW

K = 4
CHUNK = 400
NCHUNK = B_PER_W // CHUNK
NGROUP = NCHUNK // K


def _gather_body(table_hbm, idx_hbm, out_hbm, idx_v, rows, gsem, wsem):
    c = lax.axis_index("c")
    s = lax.axis_index("s")
    wid = s * NUM_CORES + c
    base = wid * B_PER_W
    pltpu.sync_copy(idx_hbm.at[pl.ds(base, B_PER_W)], idx_v)

    def gather(g, i):
        off = pl.multiple_of(g * CHUNK, CHUNK)
        pltpu.make_async_copy(
            table_hbm.at[idx_v.at[pl.ds(off, CHUNK)]], rows[i], gsem.at[i]
        ).start()

    def write(g, i):
        off = pl.multiple_of(g * CHUNK, CHUNK)
        pltpu.make_async_copy(
            rows[i], out_hbm.at[pl.ds(base + off, CHUNK)], wsem.at[i]
        ).start()

    def gather_wait(i):
        pltpu.make_async_copy(
            table_hbm.at[idx_v.at[pl.ds(0, CHUNK)]], rows[i], gsem.at[i]
        ).wait()

    def write_wait(i):
        pltpu.make_async_copy(
            rows[i], out_hbm.at[pl.ds(base, CHUNK)], wsem.at[i]
        ).wait()

    for i in range(K):
        gather(i, i)

    def body(o, carry):
        g0 = o * K
        for i in range(K):
            gather_wait(i)
            write(g0 + i, i)
        for i in range(K):
            write_wait(i)
            gather(g0 + K + i, i)
        return carry

    lax.fori_loop(0, NGROUP - 1, body, 0)

    g0 = (NGROUP - 1) * K
    for i in range(K):
        gather_wait(i)
        write(g0 + i, i)
    for i in range(K):
        write_wait(i)


@jax.jit
def _embedding_lookup(idx_flat, weight):
    mesh = plsc.VectorSubcoreMesh(core_axis_name="c", subcore_axis_name="s")
    fn = pl.kernel(
        _gather_body,
        out_type=jax.ShapeDtypeStruct((B_TOTAL, EMBED_DIM), jnp.float32),
        mesh=mesh,
        scratch_types=[
            pltpu.VMEM((B_PER_W,), jnp.int32),
            [pltpu.VMEM((CHUNK, EMBED_DIM), jnp.float32) for _ in range(K)],
            pltpu.SemaphoreType.DMA((K,)),
            pltpu.SemaphoreType.DMA((K,)),
        ],
        compiler_params=pltpu.CompilerParams(use_tc_tiling_on_sc=False),
    )
    return fn(weight, idx_flat)


def kernel(input, weight):
    idx_flat = input.reshape(-1).astype(jnp.int32)
    out = _embedding_lookup(idx_flat, weight)
    return out.reshape(BATCH, HIST, EMBED_DIM)
